# baseline (speedup 1.0000x reference)
import functools

import jax
import jax.numpy as jnp
from jax import lax
from jax.experimental import pallas as pl
from jax.experimental.pallas import tpu as pltpu
from jax.experimental.pallas import tpu_sc as plsc

BATCH = 16384
FIELDS = 26
EMBED_DIM = 16

NC = 2
NS = 16
NW = NC * NS

B = BATCH * FIELDS
B_PER_W = B // NW
CHUNK = 1664
N_CHUNKS = B_PER_W // CHUNK


@functools.partial(
    pl.kernel,
    out_type=jax.ShapeDtypeStruct((B, EMBED_DIM), jnp.float32),
    mesh=plsc.VectorSubcoreMesh(core_axis_name="c", subcore_axis_name="s"),
    scratch_types=[
        pltpu.VMEM((CHUNK,), jnp.int32),
        pltpu.VMEM((CHUNK, EMBED_DIM), jnp.float32),
        pltpu.SemaphoreType.DMA,
    ],
    compiler_params=pltpu.CompilerParams(use_tc_tiling_on_sc=False),
)
def _gather_kernel(x_hbm, table_hbm, out_hbm, idx_v, rows_v, sem):
    wid = lax.axis_index("s") * NC + lax.axis_index("c")
    for j in range(N_CHUNKS):
        base = wid * B_PER_W + j * CHUNK
        pltpu.sync_copy(x_hbm.at[pl.ds(base, CHUNK)], idx_v)
        pltpu.async_copy(table_hbm.at[idx_v], rows_v, sem).wait()
        pltpu.sync_copy(rows_v, out_hbm.at[pl.ds(base, CHUNK)])


def kernel(x, mask, table):
    del mask
    flat = _gather_kernel(x.reshape(-1), table)
    return flat.reshape(BATCH, FIELDS, EMBED_DIM)


# speedup vs baseline: 1.3495x; 1.3495x over previous
import functools

import jax
import jax.numpy as jnp
from jax import lax
from jax.experimental import pallas as pl
from jax.experimental.pallas import tpu as pltpu
from jax.experimental.pallas import tpu_sc as plsc

BATCH = 16384
FIELDS = 26
EMBED_DIM = 16

NC = 2
NS = 16
NW = NC * NS

BBLK = 128
NBLK = BATCH // BBLK
BLK_PER_W = NBLK // NW
LANES = 16


@functools.partial(
    pl.kernel,
    out_type=jax.ShapeDtypeStruct((FIELDS, EMBED_DIM, BATCH), jnp.float32),
    mesh=plsc.VectorSubcoreMesh(core_axis_name="c", subcore_axis_name="s"),
    scratch_types=[
        pltpu.VMEM((FIELDS, BBLK), jnp.int32),
        pltpu.VMEM((BBLK, EMBED_DIM), jnp.float32),
        pltpu.VMEM((EMBED_DIM, BBLK), jnp.float32),
        pltpu.SemaphoreType.DMA,
        pltpu.SemaphoreType.DMA,
    ],
    compiler_params=pltpu.CompilerParams(
        use_tc_tiling_on_sc=False, needs_layout_passes=False
    ),
)
def _embed_kernel(xt_hbm, tab_hbm, out_hbm, idx_v, rows_v, t_v, sem_g, sem_o):
    wid = lax.axis_index("s") * NC + lax.axis_index("c")
    for blk in range(BLK_PER_W):
        b0 = (wid * BLK_PER_W + blk) * BBLK
        pltpu.sync_copy(xt_hbm.at[:, pl.ds(b0, BBLK)], idx_v)

        def field_body(f, carry):
            pltpu.async_copy(tab_hbm.at[idx_v.at[f]], rows_v, sem_g).wait()
            for c in range(BBLK // LANES):
                row = lax.iota(jnp.int32, LANES) + (c * LANES)
                for d in range(EMBED_DIM):
                    col = jnp.full((LANES,), d, jnp.int32)
                    vals = plsc.load_gather(rows_v, [row, col])
                    t_v[d, pl.ds(c * LANES, LANES)] = vals
            pltpu.async_copy(t_v, out_hbm.at[f, :, pl.ds(b0, BBLK)], sem_o).wait()
            return carry

        lax.fori_loop(0, FIELDS, field_body, 0)


def kernel(x, mask, table):
    del mask
    outt = _embed_kernel(x.T, table)
    return outt.transpose(2, 0, 1)


# speedup vs baseline: 1.5400x; 1.1412x over previous
import functools

import jax
import jax.numpy as jnp
from jax import lax
from jax.experimental import pallas as pl
from jax.experimental.pallas import tpu as pltpu
from jax.experimental.pallas import tpu_sc as plsc

BATCH = 16384
FIELDS = 26
EMBED_DIM = 16

NC = 2
NS = 16
NW = NC * NS

BBLK = BATCH // NW
LANES = 16
NPAIR = FIELDS // 2


@functools.partial(
    pl.kernel,
    out_type=jax.ShapeDtypeStruct((FIELDS, EMBED_DIM, BATCH), jnp.float32),
    mesh=plsc.VectorSubcoreMesh(core_axis_name="c", subcore_axis_name="s"),
    scratch_types=[
        pltpu.VMEM((FIELDS, BBLK), jnp.int32),
        pltpu.VMEM((BBLK, EMBED_DIM), jnp.float32),
        pltpu.VMEM((BBLK, EMBED_DIM), jnp.float32),
        pltpu.VMEM((EMBED_DIM, BBLK), jnp.float32),
        pltpu.VMEM((EMBED_DIM, BBLK), jnp.float32),
        pltpu.SemaphoreType.DMA,
        pltpu.SemaphoreType.DMA,
        pltpu.SemaphoreType.DMA,
    ],
    compiler_params=pltpu.CompilerParams(
        use_tc_tiling_on_sc=False, needs_layout_passes=False
    ),
)
def _embed_kernel(xt_hbm, tab_hbm, out_hbm,
                  idx_v, rows_a, rows_b, t_a, t_b, sem_a, sem_b, sem_o):
    wid = lax.axis_index("s") * NC + lax.axis_index("c")
    b0 = wid * BBLK
    pltpu.sync_copy(xt_hbm.at[:, pl.ds(b0, BBLK)], idx_v)

    def extract(rows_v, t_v):
        for c in range(BBLK // LANES):
            row = lax.iota(jnp.int32, LANES) + (c * LANES)
            for d in range(EMBED_DIM):
                col = jnp.full((LANES,), d, jnp.int32)
                t_v[d, pl.ds(c * LANES, LANES)] = plsc.load_gather(
                    rows_v, [row, col])

    ga0 = pltpu.async_copy(tab_hbm.at[idx_v.at[0]], rows_a, sem_a)

    def pair_body(i, carry):
        f0 = 2 * i
        pltpu.make_async_copy(tab_hbm.at[idx_v.at[f0]], rows_a, sem_a).wait()
        pltpu.async_copy(tab_hbm.at[idx_v.at[f0 + 1]], rows_b, sem_b)
        extract(rows_a, t_a)
        pltpu.async_copy(t_a, out_hbm.at[f0, :, pl.ds(b0, BBLK)], sem_o).wait()
        pltpu.make_async_copy(
            tab_hbm.at[idx_v.at[f0 + 1]], rows_b, sem_b).wait()

        @pl.when(i < NPAIR - 1)
        def _():
            pltpu.async_copy(tab_hbm.at[idx_v.at[f0 + 2]], rows_a, sem_a)

        extract(rows_b, t_b)
        pltpu.async_copy(
            t_b, out_hbm.at[f0 + 1, :, pl.ds(b0, BBLK)], sem_o).wait()
        return carry

    lax.fori_loop(0, NPAIR, pair_body, 0)


def kernel(x, mask, table):
    del mask
    outt = _embed_kernel(x.T, table)
    return outt.transpose(2, 0, 1)


# speedup vs baseline: 1.7391x; 1.1293x over previous
import functools

import jax
import jax.numpy as jnp
from jax import lax
from jax.experimental import pallas as pl
from jax.experimental.pallas import tpu as pltpu
from jax.experimental.pallas import tpu_sc as plsc

BATCH = 16384
FIELDS = 26
EMBED_DIM = 16

NC = 2
NS = 16
NW = NC * NS

BBLK = BATCH // NW
LANES = 16
NPAIR = FIELDS // 2


@functools.partial(
    pl.kernel,
    out_type=jax.ShapeDtypeStruct((FIELDS, EMBED_DIM, BATCH), jnp.float32),
    mesh=plsc.VectorSubcoreMesh(core_axis_name="c", subcore_axis_name="s"),
    scratch_types=[
        pltpu.VMEM((FIELDS, BBLK), jnp.int32),
        pltpu.VMEM((BBLK, EMBED_DIM), jnp.float32),
        pltpu.VMEM((BBLK, EMBED_DIM), jnp.float32),
        pltpu.VMEM((EMBED_DIM, BBLK), jnp.float32),
        pltpu.VMEM((EMBED_DIM, BBLK), jnp.float32),
        pltpu.SemaphoreType.DMA,
        pltpu.SemaphoreType.DMA,
        pltpu.SemaphoreType.DMA,
    ],
    compiler_params=pltpu.CompilerParams(
        use_tc_tiling_on_sc=False, needs_layout_passes=False
    ),
)
def _embed_kernel(xt_hbm, tab_hbm, out_hbm,
                  idx_v, rows_a, rows_b, t_a, t_b, sem_a, sem_b, sem_o):
    wid = lax.axis_index("s") * NC + lax.axis_index("c")
    b0 = wid * BBLK
    pltpu.sync_copy(xt_hbm.at[:, pl.ds(b0, BBLK)], idx_v)

    lanes = lax.iota(jnp.int32, LANES)

    def extract(rows_v, t_v):
        def block(c, carry):
            row = lanes + c * LANES
            for k in range(EMBED_DIM):
                col = (lanes + k) & 15
                vals = plsc.load_gather(rows_v, [row, col])
                plsc.store_scatter(t_v, [col, row], vals)
            return carry

        lax.fori_loop(0, BBLK // LANES, block, 0)

    ga0 = pltpu.async_copy(tab_hbm.at[idx_v.at[0]], rows_a, sem_a)

    def pair_body(i, carry):
        f0 = 2 * i
        pltpu.make_async_copy(tab_hbm.at[idx_v.at[f0]], rows_a, sem_a).wait()
        pltpu.async_copy(tab_hbm.at[idx_v.at[f0 + 1]], rows_b, sem_b)
        extract(rows_a, t_a)
        pltpu.async_copy(t_a, out_hbm.at[f0, :, pl.ds(b0, BBLK)], sem_o).wait()
        pltpu.make_async_copy(
            tab_hbm.at[idx_v.at[f0 + 1]], rows_b, sem_b).wait()

        @pl.when(i < NPAIR - 1)
        def _():
            pltpu.async_copy(tab_hbm.at[idx_v.at[f0 + 2]], rows_a, sem_a)

        extract(rows_b, t_b)
        pltpu.async_copy(
            t_b, out_hbm.at[f0 + 1, :, pl.ds(b0, BBLK)], sem_o).wait()
        return carry

    lax.fori_loop(0, NPAIR, pair_body, 0)


def kernel(x, mask, table):
    del mask
    outt = _embed_kernel(x.T, table)
    return outt.transpose(2, 0, 1)
